# Initial kernel scaffold; baseline (speedup 1.0000x reference)
import jax
import jax.numpy as jnp
from jax.experimental import pallas as pl


def kernel(x, edge_index, batch, W1, b1, Wr, Wn, bn, W2, b2, Wlin, blin):
    raise NotImplementedError("write your pallas kernel here")


# speedup vs baseline: 14.8021x; 14.8021x over previous
import functools

import jax
import jax.numpy as jnp
from jax import lax
from jax.experimental import pallas as pl
from jax.experimental.pallas import tpu as pltpu
from jax.experimental.pallas import tpu_sc as plsc

NC, NS = 2, 16
NW = NC * NS
CHUNK = 128
RB = 1024
TB = 128


def _round_up(a, b):
    return (a + b - 1) // b * b


@functools.lru_cache(maxsize=None)
def _row_pass(NP, CH, D):
    mesh = plsc.VectorSubcoreMesh(core_axis_name="c", subcore_axis_name="s")
    rpt = NP // NS

    @functools.partial(
        pl.kernel, mesh=mesh,
        out_type=jax.ShapeDtypeStruct((NC, NP, D), jnp.float32),
        scratch_types=[
            pltpu.VMEM((CH, CHUNK), jnp.int32),
            pltpu.VMEM((CH, CHUNK), jnp.int32),
            pltpu.VMEM((CHUNK, D), jnp.float32),
            pltpu.VMEM_SHARED((NP, D), jnp.float32),
            pltpu.SemaphoreType.DMA,
        ],
    )
    def rowpass(g_hbm, src_hbm, dst_hbm, zeros_hbm, out_hbm,
                src_v, dst_v, rows_v, acc_sh, sem):
        c = lax.axis_index("c")
        s = lax.axis_index("s")
        w = c * NS + s
        r0 = s * rpt
        pltpu.sync_copy(zeros_hbm.at[pl.ds(r0, rpt)], acc_sh.at[pl.ds(r0, rpt)])
        pltpu.sync_copy(src_hbm.at[w], src_v)
        pltpu.sync_copy(dst_hbm.at[w], dst_v)
        plsc.subcore_barrier()

        def chunk(j, carry):
            pltpu.async_copy(g_hbm.at[src_v.at[j]], rows_v, sem).wait()
            pltpu.sync_copy(rows_v, acc_sh.at[dst_v.at[j]], add=True)
            return carry

        lax.fori_loop(0, CH, chunk, 0)
        plsc.subcore_barrier()
        pltpu.sync_copy(acc_sh.at[pl.ds(r0, rpt)],
                        out_hbm.at[c, pl.ds(r0, rpt)])

    return rowpass


@functools.lru_cache(maxsize=None)
def _scalar_pass(NP, CH):
    mesh = plsc.VectorSubcoreMesh(core_axis_name="c", subcore_axis_name="s")
    rpt = NP // NS

    @functools.partial(
        pl.kernel, mesh=mesh,
        out_type=jax.ShapeDtypeStruct((NC, NP), jnp.float32),
        scratch_types=[
            pltpu.VMEM((CH, CHUNK), jnp.int32),
            pltpu.VMEM((CH, CHUNK), jnp.int32),
            pltpu.VMEM((CHUNK,), jnp.float32),
            pltpu.VMEM_SHARED((NP,), jnp.float32),
            pltpu.SemaphoreType.DMA,
        ],
    )
    def scpass(val_hbm, src_hbm, dst_hbm, zeros_hbm, out_hbm,
               src_v, dst_v, vals_v, acc_sh, sem):
        c = lax.axis_index("c")
        s = lax.axis_index("s")
        w = c * NS + s
        r0 = s * rpt
        pltpu.sync_copy(zeros_hbm.at[pl.ds(r0, rpt)], acc_sh.at[pl.ds(r0, rpt)])
        pltpu.sync_copy(src_hbm.at[w], src_v)
        pltpu.sync_copy(dst_hbm.at[w], dst_v)
        plsc.subcore_barrier()

        def chunk(j, carry):
            pltpu.async_copy(val_hbm.at[src_v.at[j]], vals_v, sem).wait()
            pltpu.sync_copy(vals_v, acc_sh.at[dst_v.at[j]], add=True)
            return carry

        lax.fori_loop(0, CH, chunk, 0)
        plsc.subcore_barrier()
        pltpu.sync_copy(acc_sh.at[pl.ds(r0, rpt)],
                        out_hbm.at[c, pl.ds(r0, rpt)])

    return scpass


def _pre_body(x_ref, w_ref, degp_ref, hpre_ref, g1_ref, dinv_ref):
    hpre = jnp.dot(x_ref[...], w_ref[...], preferred_element_type=jnp.float32)
    deg = degp_ref[0] + degp_ref[1] + 1.0
    dinv = lax.rsqrt(deg)
    hpre_ref[...] = hpre
    g1_ref[...] = hpre * dinv
    dinv_ref[...] = dinv


def _post1_body(accp_ref, hpre_ref, dinv_ref, b1_ref, wr_ref,
                h_ref, hr_ref, *, n_valid):
    i = pl.program_id(0)
    acc = accp_ref[0] + accp_ref[1]
    dinv = dinv_ref[...]
    hraw = jax.nn.relu(dinv * acc + dinv * dinv * hpre_ref[...] + b1_ref[...])
    rows = i * RB + lax.broadcasted_iota(jnp.int32, (RB, 1), 0)
    h = jnp.where(rows < n_valid, hraw, 0.0)
    h_ref[...] = h
    hr_ref[...] = jnp.dot(h, wr_ref[...], preferred_element_type=jnp.float32)


def _score_body(sp_ref, hr_ref, wn_ref, bn_ref, score_ref, t_ref):
    s_rows = sp_ref[0] + sp_ref[1]
    sc = (jnp.dot(s_rows, wn_ref[...], preferred_element_type=jnp.float32)
          + bn_ref[...]) + hr_ref[...]
    score_ref[...] = sc
    t_ref[...] = jnp.tanh(sc)


def _topk_body(scol_ref, bcol_ref, srow_ref, brow_ref, jlo_ref, jhi_ref,
               keep_ref, *, n_valid):
    I = pl.program_id(0)
    si = scol_ref[...]
    bi = bcol_ref[...]
    ii = I * TB + lax.broadcasted_iota(jnp.int32, (TB, 1), 0)

    def body(j, carry):
        rank, cnt = carry
        sj = srow_ref[pl.ds(j, 1), :]
        bj = brow_ref[pl.ds(j, 1), :]
        jj = j * 128 + lax.broadcasted_iota(jnp.int32, (1, 128), 1)
        same = bi == bj
        before = (sj > si) | ((sj == si) & (jj < ii))
        rank = rank + jnp.sum((same & before).astype(jnp.int32), axis=1,
                              keepdims=True)
        cnt = cnt + jnp.sum(same.astype(jnp.int32), axis=1, keepdims=True)
        return rank, cnt

    z = jnp.zeros((TB, 1), jnp.int32)
    rank, cnt = lax.fori_loop(jlo_ref[I], jhi_ref[I] + 1, body, (z, z))
    k = (cnt + 1) // 2
    keep_ref[...] = ((rank < k) & (ii < n_valid)).astype(jnp.float32)


def _mid2_body(h_ref, t_ref, keep_ref, cntp_ref, w2_ref,
               hpre2_ref, g2_ref, d2k_ref, d2sq_ref):
    keep = keep_ref[...]
    deg2 = keep * (cntp_ref[0] + cntp_ref[1]) + 1.0
    dinv2 = lax.rsqrt(deg2)
    h2in = h_ref[...] * t_ref[...] * keep
    hpre2 = jnp.dot(h2in, w2_ref[...], preferred_element_type=jnp.float32)
    hpre2_ref[...] = hpre2
    g2_ref[...] = hpre2 * dinv2
    d2k_ref[...] = dinv2 * keep
    d2sq_ref[...] = dinv2 * dinv2


def _final_body(accp_ref, hpre2_ref, d2k_ref, d2sq_ref, b2_ref, wlin_ref,
                blin_ref, keep_ref, h3_ref):
    acc = accp_ref[0] + accp_ref[1]
    out2 = jax.nn.relu(d2k_ref[...] * acc + d2sq_ref[...] * hpre2_ref[...]
                       + b2_ref[...])
    h3 = jnp.dot(out2, wlin_ref[...], preferred_element_type=jnp.float32)
    h3_ref[...] = (h3 + blin_ref[...]) * keep_ref[...]


def _blk(shape, imap):
    return pl.BlockSpec(shape, imap)


def kernel(x, edge_index, batch, W1, b1, Wr, Wn, bn, W2, b2, Wlin, blin):
    f32 = jnp.float32
    N, D = x.shape
    H = W1.shape[1]
    E = edge_index.shape[1]
    NP = _round_up(N + 1, RB)
    EP = _round_up(E, NW * CHUNK)
    CH = EP // (NW * CHUNK)
    NB = NP // RB
    NT = NP // TB

    xp = jnp.pad(x, ((0, NP - N), (0, 0)))
    src = edge_index[0]
    dst = edge_index[1]
    srcp = jnp.concatenate(
        [src, jnp.full((EP - E,), N, jnp.int32)]).reshape(NW, CH, CHUNK)
    dstp = jnp.concatenate(
        [dst, jnp.zeros((EP - E,), jnp.int32)]).reshape(NW, CH, CHUNK)
    zeros_rows = jnp.zeros((NP, H), f32)
    zeros_vec = jnp.zeros((NP,), f32)
    ones_val = jnp.pad(jnp.ones((N,), f32), (0, NP - N))

    scal = _scalar_pass(NP, CH)
    rowp = _row_pass(NP, CH, H)

    degp = scal(ones_val, srcp, dstp, zeros_vec)
    hpre, g1, dinv = pl.pallas_call(
        _pre_body,
        grid=(NB,),
        in_specs=[
            _blk((RB, D), lambda i: (i, 0)),
            _blk((D, H), lambda i: (0, 0)),
            _blk((NC, RB, 1), lambda i: (0, i, 0)),
        ],
        out_specs=[
            _blk((RB, H), lambda i: (i, 0)),
            _blk((RB, H), lambda i: (i, 0)),
            _blk((RB, 1), lambda i: (i, 0)),
        ],
        out_shape=[
            jax.ShapeDtypeStruct((NP, H), f32),
            jax.ShapeDtypeStruct((NP, H), f32),
            jax.ShapeDtypeStruct((NP, 1), f32),
        ],
    )(xp, W1, degp.reshape(NC, NP, 1))

    acc1p = rowp(g1, srcp, dstp, zeros_rows)
    h, hr = pl.pallas_call(
        functools.partial(_post1_body, n_valid=N),
        grid=(NB,),
        in_specs=[
            _blk((NC, RB, H), lambda i: (0, i, 0)),
            _blk((RB, H), lambda i: (i, 0)),
            _blk((RB, 1), lambda i: (i, 0)),
            _blk((1, H), lambda i: (0, 0)),
            _blk((H, 1), lambda i: (0, 0)),
        ],
        out_specs=[
            _blk((RB, H), lambda i: (i, 0)),
            _blk((RB, 1), lambda i: (i, 0)),
        ],
        out_shape=[
            jax.ShapeDtypeStruct((NP, H), f32),
            jax.ShapeDtypeStruct((NP, 1), f32),
        ],
    )(acc1p, hpre, dinv, b1.reshape(1, H), Wr)

    sp = rowp(h, srcp, dstp, zeros_rows)
    score, t = pl.pallas_call(
        _score_body,
        grid=(NB,),
        in_specs=[
            _blk((NC, RB, H), lambda i: (0, i, 0)),
            _blk((RB, 1), lambda i: (i, 0)),
            _blk((H, 1), lambda i: (0, 0)),
            _blk((1, 1), lambda i: (0, 0)),
        ],
        out_specs=[
            _blk((RB, 1), lambda i: (i, 0)),
            _blk((RB, 1), lambda i: (i, 0)),
        ],
        out_shape=[
            jax.ShapeDtypeStruct((NP, 1), f32),
            jax.ShapeDtypeStruct((NP, 1), f32),
        ],
    )(sp, hr, Wn, bn.reshape(1, 1))

    batch_p = jnp.concatenate(
        [batch, jnp.full((NP - N,), jnp.int32(1 << 20))])
    r0 = jnp.arange(NT, dtype=jnp.int32) * TB
    lo_g = batch_p[r0]
    hi_g = batch_p[r0 + TB - 1]
    jlo = (jnp.searchsorted(batch_p, lo_g, side="left") // 128).astype(jnp.int32)
    jhi = ((jnp.searchsorted(batch_p, hi_g, side="right") - 1) // 128).astype(jnp.int32)
    keep = pl.pallas_call(
        functools.partial(_topk_body, n_valid=N),
        grid=(NT,),
        in_specs=[
            _blk((TB, 1), lambda i: (i, 0)),
            _blk((TB, 1), lambda i: (i, 0)),
            _blk((NP // 128, 128), lambda i: (0, 0)),
            _blk((NP // 128, 128), lambda i: (0, 0)),
            pl.BlockSpec(memory_space=pltpu.SMEM),
            pl.BlockSpec(memory_space=pltpu.SMEM),
        ],
        out_specs=_blk((TB, 1), lambda i: (i, 0)),
        out_shape=jax.ShapeDtypeStruct((NP, 1), f32),
    )(score, batch_p.reshape(NP, 1), score.reshape(NP // 128, 128),
      batch_p.reshape(NP // 128, 128), jlo, jhi)

    cntp = scal(keep.reshape(NP), srcp, dstp, zeros_vec)
    hpre2, g2, d2k, d2sq = pl.pallas_call(
        _mid2_body,
        grid=(NB,),
        in_specs=[
            _blk((RB, H), lambda i: (i, 0)),
            _blk((RB, 1), lambda i: (i, 0)),
            _blk((RB, 1), lambda i: (i, 0)),
            _blk((NC, RB, 1), lambda i: (0, i, 0)),
            _blk((H, H), lambda i: (0, 0)),
        ],
        out_specs=[
            _blk((RB, H), lambda i: (i, 0)),
            _blk((RB, H), lambda i: (i, 0)),
            _blk((RB, 1), lambda i: (i, 0)),
            _blk((RB, 1), lambda i: (i, 0)),
        ],
        out_shape=[
            jax.ShapeDtypeStruct((NP, H), f32),
            jax.ShapeDtypeStruct((NP, H), f32),
            jax.ShapeDtypeStruct((NP, 1), f32),
            jax.ShapeDtypeStruct((NP, 1), f32),
        ],
    )(h, t, keep, cntp.reshape(NC, NP, 1), W2)

    acc2p = rowp(g2, srcp, dstp, zeros_rows)
    h3 = pl.pallas_call(
        _final_body,
        grid=(NB,),
        in_specs=[
            _blk((NC, RB, H), lambda i: (0, i, 0)),
            _blk((RB, H), lambda i: (i, 0)),
            _blk((RB, 1), lambda i: (i, 0)),
            _blk((RB, 1), lambda i: (i, 0)),
            _blk((1, H), lambda i: (0, 0)),
            _blk((H, H), lambda i: (0, 0)),
            _blk((1, H), lambda i: (0, 0)),
            _blk((RB, 1), lambda i: (i, 0)),
        ],
        out_specs=_blk((RB, H), lambda i: (i, 0)),
        out_shape=jax.ShapeDtypeStruct((NP, H), f32),
    )(acc2p, hpre2, d2k, d2sq, b2.reshape(1, H), Wlin, blin.reshape(1, H),
      keep)

    return h3[:N], batch
